# initial kernel scaffold (unmeasured)
import jax
import jax.numpy as jnp
from jax import lax
from jax.experimental import pallas as pl
from jax.experimental.pallas import tpu as pltpu

N_DEV = 8


def kernel(x, w_mat):
    m_per, k = x.shape
    _, n_per = w_mat.shape
    m_tot = N_DEV * m_per

    def body(x_ref, w_ref, out_ref, xg_ref, y_ref, ax_ref,
             ring_send, ring_recv, ax_send, ax_recv):
        my = lax.axis_index("i")
        left = lax.rem(my + N_DEV - 1, N_DEV)
        right = lax.rem(my + 1, N_DEV)

        barrier = pltpu.get_barrier_semaphore()
        for nbr in (left, right):
            pl.semaphore_signal(barrier, inc=1, device_id=(nbr,),
                                device_id_type=pl.DeviceIdType.MESH)
        pl.semaphore_wait(barrier, 2)

        xg_ref[pl.ds(my * m_per, m_per), :] = x_ref[...]

        def gemm_block(origin):
            chunk = xg_ref[pl.ds(origin * m_per, m_per), :]
            yb = jnp.dot(chunk, w_ref[...], preferred_element_type=jnp.float32)
            yb = jnp.maximum(yb, 0.0)
            y_ref[pl.ds(origin * m_per, m_per), :] = yb
            return jnp.max(yb)

        amax = gemm_block(my)

        for h in range(N_DEV - 1):
            s_org = lax.rem(my + N_DEV - h, N_DEV)
            r_org = lax.rem(my + N_DEV - h - 1, N_DEV)
            rdma = pltpu.make_async_remote_copy(
                src_ref=xg_ref.at[pl.ds(s_org * m_per, m_per), :],
                dst_ref=xg_ref.at[pl.ds(s_org * m_per, m_per), :],
                send_sem=ring_send.at[h],
                recv_sem=ring_recv.at[h],
                device_id=(right,),
                device_id_type=pl.DeviceIdType.MESH,
            )
            rdma.start()
            rdma.wait()
            amax = jnp.maximum(amax, gemm_block(r_org))

        ax_ref[pl.ds(my, 1), :] = jnp.full((1, 128), amax, jnp.float32)
        rdmas = []
        for d in range(1, N_DEV):
            p = lax.rem(my + d, N_DEV)
            r = pltpu.make_async_remote_copy(
                src_ref=ax_ref.at[pl.ds(my, 1), :],
                dst_ref=ax_ref.at[pl.ds(my, 1), :],
                send_sem=ax_send.at[d - 1],
                recv_sem=ax_recv.at[d - 1],
                device_id=(p,),
                device_id_type=pl.DeviceIdType.MESH,
            )
            r.start()
            rdmas.append(r)
        for r in rdmas:
            r.wait()

        amax_g = jnp.maximum(jnp.max(ax_ref[...]), 1e-30)
        inv_scale = 127.0 / amax_g
        q = jnp.clip(jnp.round(y_ref[...] * inv_scale), -127.0, 127.0)
        out_ref[...] = q.astype(jnp.int8)

    return pl.pallas_call(
        body,
        out_shape=jax.ShapeDtypeStruct((m_tot, n_per), jnp.int8),
        in_specs=[
            pl.BlockSpec(memory_space=pltpu.VMEM),
            pl.BlockSpec(memory_space=pltpu.VMEM),
        ],
        out_specs=pl.BlockSpec(memory_space=pltpu.VMEM),
        scratch_shapes=[
            pltpu.VMEM((m_tot, k), jnp.float32),
            pltpu.VMEM((m_tot, n_per), jnp.float32),
            pltpu.VMEM((N_DEV, 128), jnp.float32),
            pltpu.SemaphoreType.DMA((N_DEV - 1,)),
            pltpu.SemaphoreType.DMA((N_DEV - 1,)),
            pltpu.SemaphoreType.DMA((N_DEV - 1,)),
            pltpu.SemaphoreType.DMA((N_DEV - 1,)),
        ],
        compiler_params=pltpu.CompilerParams(
            collective_id=0,
            vmem_limit_bytes=128 * 1024 * 1024,
        ),
    )(x, w_mat)


# baseline (device time: 725159 ns/iter reference)
import jax
import jax.numpy as jnp
from jax import lax
from jax.experimental import pallas as pl
from jax.experimental.pallas import tpu as pltpu

N_DEV = 8


def kernel(x, w_mat):
    m_per, k = x.shape
    _, n_per = w_mat.shape
    m_tot = N_DEV * m_per

    def body(x_ref, w_ref, out_ref, comm_ref, ax_ref,
             ring_send, ring_recv, credit_sem, ax_send, ax_recv):
        my = lax.axis_index("i")
        left = lax.rem(my + N_DEV - 1, N_DEV)
        right = lax.rem(my + 1, N_DEV)

        barrier = pltpu.get_barrier_semaphore()
        for nbr in (left, right):
            pl.semaphore_signal(barrier, inc=1, device_id=(nbr,),
                                device_id_type=pl.DeviceIdType.MESH)
        pl.semaphore_wait(barrier, 2)

        def gemm_block(chunk, origin):
            yb = jnp.dot(chunk, w_ref[...], preferred_element_type=jnp.float32)
            yb = jnp.maximum(yb, 0.0)
            out_ref[pl.ds(origin * m_per, m_per), :] = yb
            return jnp.max(yb)

        amax = gemm_block(x_ref[...], my)

        for h in range(N_DEV - 1):
            if h >= 2:
                pl.semaphore_wait(credit_sem, 1)
            src = x_ref if h == 0 else comm_ref.at[h % 2]
            rdma = pltpu.make_async_remote_copy(
                src_ref=src,
                dst_ref=comm_ref.at[(h + 1) % 2],
                send_sem=ring_send.at[h],
                recv_sem=ring_recv.at[h],
                device_id=(right,),
                device_id_type=pl.DeviceIdType.MESH,
            )
            rdma.start()
            rdma.wait()
            r_org = lax.rem(my + N_DEV - h - 1, N_DEV)
            amax = jnp.maximum(amax, gemm_block(comm_ref[(h + 1) % 2], r_org))
            if h <= 4:
                pl.semaphore_signal(credit_sem, inc=1, device_id=(left,),
                                    device_id_type=pl.DeviceIdType.MESH)

        ax_ref[pl.ds(my, 1), :] = jnp.full((1, 128), amax, jnp.float32)
        rdmas = []
        for d in range(1, N_DEV):
            p = lax.rem(my + d, N_DEV)
            r = pltpu.make_async_remote_copy(
                src_ref=ax_ref.at[pl.ds(my, 1), :],
                dst_ref=ax_ref.at[pl.ds(my, 1), :],
                send_sem=ax_send.at[d - 1],
                recv_sem=ax_recv.at[d - 1],
                device_id=(p,),
                device_id_type=pl.DeviceIdType.MESH,
            )
            r.start()
            rdmas.append(r)
        for r in rdmas:
            r.wait()

        amax_g = jnp.maximum(jnp.max(ax_ref[...]), 1e-30)
        scale = amax_g / 127.0
        inv_scale = 127.0 / amax_g
        for b in range(N_DEV):
            yb = out_ref[b * m_per:(b + 1) * m_per, :]
            q = jnp.clip(jnp.round(yb * inv_scale), -127.0, 127.0)
            out_ref[b * m_per:(b + 1) * m_per, :] = q * scale

    return pl.pallas_call(
        body,
        out_shape=jax.ShapeDtypeStruct((m_tot, n_per), jnp.float32),
        in_specs=[
            pl.BlockSpec(memory_space=pltpu.VMEM),
            pl.BlockSpec(memory_space=pltpu.VMEM),
        ],
        out_specs=pl.BlockSpec(memory_space=pltpu.VMEM),
        scratch_shapes=[
            pltpu.VMEM((2, m_per, k), jnp.float32),
            pltpu.VMEM((N_DEV, 128), jnp.float32),
            pltpu.SemaphoreType.DMA((N_DEV - 1,)),
            pltpu.SemaphoreType.DMA((N_DEV - 1,)),
            pltpu.SemaphoreType.REGULAR,
            pltpu.SemaphoreType.DMA((N_DEV - 1,)),
            pltpu.SemaphoreType.DMA((N_DEV - 1,)),
        ],
        compiler_params=pltpu.CompilerParams(
            collective_id=0,
            vmem_limit_bytes=64 * 1024 * 1024,
        ),
    )(x, w_mat)


# device time: 378265 ns/iter; 1.9171x vs baseline; 1.9171x over previous
import jax
import jax.numpy as jnp
from jax import lax
from jax.experimental import pallas as pl
from jax.experimental.pallas import tpu as pltpu

N_DEV = 8


def kernel(x, w_mat):
    m_per, k = x.shape
    _, n_per = w_mat.shape
    m_tot = N_DEV * m_per
    hm = m_per // 2

    def body(x_ref, w_ref, out_ref, comm_r, comm_l, ax_ref,
             send_r, recv_r, send_l, recv_l, credit_r, credit_l,
             ax_send, ax_recv):
        my = lax.axis_index("i")
        left = lax.rem(my + N_DEV - 1, N_DEV)
        right = lax.rem(my + 1, N_DEV)

        barrier = pltpu.get_barrier_semaphore()
        for nbr in (left, right):
            pl.semaphore_signal(barrier, inc=1, device_id=(nbr,),
                                device_id_type=pl.DeviceIdType.MESH)
        pl.semaphore_wait(barrier, 2)

        def gemm(chunk, row0, nrows):
            yb = jnp.dot(chunk, w_ref[...], preferred_element_type=jnp.float32)
            yb = jnp.maximum(yb, 0.0)
            out_ref[pl.ds(row0, nrows), :] = yb
            return jnp.max(yb)

        amax = jnp.float32(0.0)

        for h in range(N_DEV - 1):
            if h >= 2:
                pl.semaphore_wait(credit_r, 1)
                pl.semaphore_wait(credit_l, 1)
            src_r = x_ref.at[pl.ds(0, hm), :] if h == 0 else comm_r.at[h % 2]
            src_l = x_ref.at[pl.ds(hm, hm), :] if h == 0 else comm_l.at[h % 2]
            rdma_r = pltpu.make_async_remote_copy(
                src_ref=src_r,
                dst_ref=comm_r.at[(h + 1) % 2],
                send_sem=send_r.at[h],
                recv_sem=recv_r.at[h],
                device_id=(right,),
                device_id_type=pl.DeviceIdType.MESH,
            )
            rdma_l = pltpu.make_async_remote_copy(
                src_ref=src_l,
                dst_ref=comm_l.at[(h + 1) % 2],
                send_sem=send_l.at[h],
                recv_sem=recv_l.at[h],
                device_id=(left,),
                device_id_type=pl.DeviceIdType.MESH,
            )
            rdma_r.start()
            rdma_l.start()

            if h == 0:
                amax = jnp.maximum(amax, gemm(x_ref[...], my * m_per, m_per))
            else:
                org_r = lax.rem(my + N_DEV - h, N_DEV)
                org_l = lax.rem(my + h, N_DEV)
                amax = jnp.maximum(
                    amax, gemm(comm_r[h % 2], org_r * m_per, hm))
                amax = jnp.maximum(
                    amax, gemm(comm_l[h % 2], org_l * m_per + hm, hm))

            rdma_r.wait()
            rdma_l.wait()
            if 1 <= h <= 5:
                pl.semaphore_signal(credit_r, inc=1, device_id=(left,),
                                    device_id_type=pl.DeviceIdType.MESH)
                pl.semaphore_signal(credit_l, inc=1, device_id=(right,),
                                    device_id_type=pl.DeviceIdType.MESH)

        org_r = lax.rem(my + 1, N_DEV)
        org_l = lax.rem(my + N_DEV - 1, N_DEV)
        amax = jnp.maximum(amax, gemm(comm_r[1], org_r * m_per, hm))
        amax = jnp.maximum(amax, gemm(comm_l[1], org_l * m_per + hm, hm))

        ax_ref[pl.ds(my, 1), :] = jnp.full((1, 128), amax, jnp.float32)
        rdmas = []
        for d in range(1, N_DEV):
            p = lax.rem(my + d, N_DEV)
            r = pltpu.make_async_remote_copy(
                src_ref=ax_ref.at[pl.ds(my, 1), :],
                dst_ref=ax_ref.at[pl.ds(my, 1), :],
                send_sem=ax_send.at[d - 1],
                recv_sem=ax_recv.at[d - 1],
                device_id=(p,),
                device_id_type=pl.DeviceIdType.MESH,
            )
            r.start()
            rdmas.append(r)
        for r in rdmas:
            r.wait()

        amax_g = jnp.maximum(jnp.max(ax_ref[...]), 1e-30)
        scale = amax_g / 127.0
        inv_scale = 127.0 / amax_g
        for b in range(N_DEV):
            yb = out_ref[b * m_per:(b + 1) * m_per, :]
            q = jnp.clip(jnp.round(yb * inv_scale), -127.0, 127.0)
            out_ref[b * m_per:(b + 1) * m_per, :] = q * scale

    return pl.pallas_call(
        body,
        out_shape=jax.ShapeDtypeStruct((m_tot, n_per), jnp.float32),
        in_specs=[
            pl.BlockSpec(memory_space=pltpu.VMEM),
            pl.BlockSpec(memory_space=pltpu.VMEM),
        ],
        out_specs=pl.BlockSpec(memory_space=pltpu.VMEM),
        scratch_shapes=[
            pltpu.VMEM((2, hm, k), jnp.float32),
            pltpu.VMEM((2, hm, k), jnp.float32),
            pltpu.VMEM((N_DEV, 128), jnp.float32),
            pltpu.SemaphoreType.DMA((N_DEV - 1,)),
            pltpu.SemaphoreType.DMA((N_DEV - 1,)),
            pltpu.SemaphoreType.DMA((N_DEV - 1,)),
            pltpu.SemaphoreType.DMA((N_DEV - 1,)),
            pltpu.SemaphoreType.REGULAR,
            pltpu.SemaphoreType.REGULAR,
            pltpu.SemaphoreType.DMA((N_DEV - 1,)),
            pltpu.SemaphoreType.DMA((N_DEV - 1,)),
        ],
        compiler_params=pltpu.CompilerParams(
            collective_id=0,
            vmem_limit_bytes=64 * 1024 * 1024,
        ),
    )(x, w_mat)


# device time: 221430 ns/iter; 3.2749x vs baseline; 1.7083x over previous
import jax
import jax.numpy as jnp
from jax import lax
from jax.experimental import pallas as pl
from jax.experimental.pallas import tpu as pltpu

N_DEV = 8


def kernel(x, w_mat):
    m_per, k = x.shape
    _, n_per = w_mat.shape
    m_tot = N_DEV * m_per
    hm = m_per // 2

    def body(x_ref, w_ref, out_ref, comm_r, comm_l, xb_ref, ax_ref,
             send_r, recv_r, send_l, recv_l, credit_r, credit_l,
             ax_send, ax_recv):
        my = lax.axis_index("i")
        left = lax.rem(my + N_DEV - 1, N_DEV)
        right = lax.rem(my + 1, N_DEV)

        barrier = pltpu.get_barrier_semaphore()
        for nbr in (left, right):
            pl.semaphore_signal(barrier, inc=1, device_id=(nbr,),
                                device_id_type=pl.DeviceIdType.MESH)
        pl.semaphore_wait(barrier, 2)

        def gemm(chunk, row0, nrows):
            yb = jnp.dot(chunk, w_ref[...], preferred_element_type=jnp.float32)
            yb = jnp.maximum(yb, 0.0)
            out_ref[pl.ds(row0, nrows), :] = yb
            return jnp.max(yb)

        xb_ref[...] = x_ref[...].astype(jnp.bfloat16)

        amax = jnp.float32(0.0)

        for h in range(N_DEV - 1):
            if h >= 2:
                pl.semaphore_wait(credit_r, 1)
                pl.semaphore_wait(credit_l, 1)
            src_r = xb_ref.at[pl.ds(0, hm), :] if h == 0 else comm_r.at[h % 2]
            src_l = xb_ref.at[pl.ds(hm, hm), :] if h == 0 else comm_l.at[h % 2]
            rdma_r = pltpu.make_async_remote_copy(
                src_ref=src_r,
                dst_ref=comm_r.at[(h + 1) % 2],
                send_sem=send_r.at[h],
                recv_sem=recv_r.at[h],
                device_id=(right,),
                device_id_type=pl.DeviceIdType.MESH,
            )
            rdma_l = pltpu.make_async_remote_copy(
                src_ref=src_l,
                dst_ref=comm_l.at[(h + 1) % 2],
                send_sem=send_l.at[h],
                recv_sem=recv_l.at[h],
                device_id=(left,),
                device_id_type=pl.DeviceIdType.MESH,
            )
            rdma_r.start()
            rdma_l.start()

            if h == 0:
                amax = jnp.maximum(amax, gemm(x_ref[...], my * m_per, m_per))
            else:
                org_r = lax.rem(my + N_DEV - h, N_DEV)
                org_l = lax.rem(my + h, N_DEV)
                amax = jnp.maximum(
                    amax, gemm(comm_r[h % 2], org_r * m_per, hm))
                amax = jnp.maximum(
                    amax, gemm(comm_l[h % 2], org_l * m_per + hm, hm))

            rdma_r.wait()
            rdma_l.wait()
            if 1 <= h <= 5:
                pl.semaphore_signal(credit_r, inc=1, device_id=(left,),
                                    device_id_type=pl.DeviceIdType.MESH)
                pl.semaphore_signal(credit_l, inc=1, device_id=(right,),
                                    device_id_type=pl.DeviceIdType.MESH)

        org_r = lax.rem(my + 1, N_DEV)
        org_l = lax.rem(my + N_DEV - 1, N_DEV)
        amax = jnp.maximum(amax, gemm(comm_r[1], org_r * m_per, hm))
        amax = jnp.maximum(amax, gemm(comm_l[1], org_l * m_per + hm, hm))

        ax_ref[pl.ds(my, 1), :] = jnp.full((1, 128), amax, jnp.float32)
        rdmas = []
        for d in range(1, N_DEV):
            p = lax.rem(my + d, N_DEV)
            r = pltpu.make_async_remote_copy(
                src_ref=ax_ref.at[pl.ds(my, 1), :],
                dst_ref=ax_ref.at[pl.ds(my, 1), :],
                send_sem=ax_send.at[d - 1],
                recv_sem=ax_recv.at[d - 1],
                device_id=(p,),
                device_id_type=pl.DeviceIdType.MESH,
            )
            r.start()
            rdmas.append(r)
        for r in rdmas:
            r.wait()

        amax_g = jnp.maximum(jnp.max(ax_ref[...]), 1e-30)
        scale = amax_g / 127.0
        inv_scale = 127.0 / amax_g
        for b in range(N_DEV):
            yb = out_ref[b * m_per:(b + 1) * m_per, :]
            q = jnp.clip(jnp.round(yb * inv_scale), -127.0, 127.0)
            out_ref[b * m_per:(b + 1) * m_per, :] = q * scale

    return pl.pallas_call(
        body,
        out_shape=jax.ShapeDtypeStruct((m_tot, n_per), jnp.float32),
        in_specs=[
            pl.BlockSpec(memory_space=pltpu.VMEM),
            pl.BlockSpec(memory_space=pltpu.VMEM),
        ],
        out_specs=pl.BlockSpec(memory_space=pltpu.VMEM),
        scratch_shapes=[
            pltpu.VMEM((2, hm, k), jnp.bfloat16),
            pltpu.VMEM((2, hm, k), jnp.bfloat16),
            pltpu.VMEM((m_per, k), jnp.bfloat16),
            pltpu.VMEM((N_DEV, 128), jnp.float32),
            pltpu.SemaphoreType.DMA((N_DEV - 1,)),
            pltpu.SemaphoreType.DMA((N_DEV - 1,)),
            pltpu.SemaphoreType.DMA((N_DEV - 1,)),
            pltpu.SemaphoreType.DMA((N_DEV - 1,)),
            pltpu.SemaphoreType.REGULAR,
            pltpu.SemaphoreType.REGULAR,
            pltpu.SemaphoreType.DMA((N_DEV - 1,)),
            pltpu.SemaphoreType.DMA((N_DEV - 1,)),
        ],
        compiler_params=pltpu.CompilerParams(
            collective_id=0,
            vmem_limit_bytes=64 * 1024 * 1024,
        ),
    )(x, w_mat)
